# Initial kernel scaffold; baseline (speedup 1.0000x reference)
import jax
import jax.numpy as jnp
from jax.experimental import pallas as pl


def kernel(x, edge_index, W1, b1, W2, b2, W3, b3):
    raise NotImplementedError("write your pallas kernel here")


# speedup vs baseline: 13.6378x; 13.6378x over previous
import functools

import jax
import jax.numpy as jnp
from jax import lax
from jax.experimental import pallas as pl
from jax.experimental.pallas import tpu as pltpu
from jax.experimental.pallas import tpu_sc as plsc

NC = 2
NS = 16
NW = NC * NS
CHUNK = 128
LANES = 16


def _cdiv(a, b):
    return (a + b - 1) // b


def _make_deg_kernel(nchunks, n_pad, rpt):
    mesh = plsc.VectorSubcoreMesh(core_axis_name="c", subcore_axis_name="s")

    @functools.partial(
        pl.kernel,
        out_type=jax.ShapeDtypeStruct((NC, n_pad), jnp.float32),
        mesh=mesh,
        scratch_types=[
            pltpu.VMEM((nchunks, CHUNK), jnp.int32),
            pltpu.VMEM((CHUNK,), jnp.float32),
            pltpu.VMEM((rpt,), jnp.float32),
            pltpu.VMEM_SHARED((n_pad,), jnp.float32),
        ],
    )
    def deg_kernel(dst_hbm, out_hbm, dst_v, ones_v, zbuf, acc):
        cid = lax.axis_index("c")
        sid = lax.axis_index("s")
        wid = sid * NC + cid

        def fill_z(j, _):
            zbuf[pl.ds(j * LANES, LANES)] = jnp.zeros((LANES,), jnp.float32)
            return 0

        lax.fori_loop(0, rpt // LANES, fill_z, 0)
        for j in range(CHUNK // LANES):
            ones_v[pl.ds(j * LANES, LANES)] = jnp.ones((LANES,), jnp.float32)
        pltpu.sync_copy(zbuf, acc.at[pl.ds(sid * rpt, rpt)])
        pltpu.sync_copy(dst_hbm.at[wid], dst_v)
        plsc.subcore_barrier()

        def chunk_step(c, _):
            pltpu.sync_copy(ones_v, acc.at[dst_v.at[c]], add=True)
            return 0

        lax.fori_loop(0, nchunks, chunk_step, 0)
        plsc.subcore_barrier()
        pltpu.sync_copy(acc.at[pl.ds(sid * rpt, rpt)],
                        out_hbm.at[cid, pl.ds(sid * rpt, rpt)])

    return deg_kernel


def _make_edge_kernel(n, d, nchunks, n_pad, rpt, nbuf=2, group=16):
    mesh = plsc.VectorSubcoreMesh(core_axis_name="c", subcore_axis_name="s")
    scratch = [
        pltpu.VMEM((group, CHUNK), jnp.int32),
        pltpu.VMEM((group, CHUNK), jnp.int32),
        pltpu.VMEM_SHARED((n_pad, d), jnp.float32),
    ]
    scratch += [pltpu.VMEM((CHUNK, d), jnp.float32) for _ in range(nbuf)]
    scratch += [pltpu.SemaphoreType.DMA for _ in range(nbuf)]

    @functools.partial(
        pl.kernel,
        out_type=jax.ShapeDtypeStruct((NC, n_pad, d), jnp.float32),
        mesh=mesh,
        scratch_types=scratch,
    )
    def edge_kernel(y_hbm, src_hbm, dst_hbm, out_hbm, src_v, dst_v, acc, *rest):
        bufs = rest[:nbuf]
        sems = rest[nbuf:]
        cid = lax.axis_index("c")
        sid = lax.axis_index("s")
        wid = sid * NC + cid

        def zrow(r, _):
            for j in range(d // LANES):
                bufs[0][r, pl.ds(j * LANES, LANES)] = jnp.zeros(
                    (LANES,), jnp.float32)
            return 0

        lax.fori_loop(0, CHUNK, zrow, 0)
        for t in range(rpt // CHUNK):
            pltpu.sync_copy(
                bufs[0], acc.at[pl.ds(sid * rpt + t * CHUNK, CHUNK)])
        plsc.subcore_barrier()

        for s in range(_cdiv(nchunks, group)):
            lo = s * group
            cnt = min(group, nchunks - lo)
            pltpu.sync_copy(src_hbm.at[wid, pl.ds(lo, cnt)],
                            src_v.at[pl.ds(0, cnt)])
            pltpu.sync_copy(dst_hbm.at[wid, pl.ds(lo, cnt)],
                            dst_v.at[pl.ds(0, cnt)])
            for b in range(min(nbuf, cnt)):
                pltpu.async_copy(y_hbm.at[src_v.at[b]], bufs[b], sems[b])

            def ring(g, _):
                for b in range(nbuf):
                    j = g * nbuf + b

                    @pl.when(j < cnt)
                    def _():
                        pltpu.make_async_copy(
                            y_hbm.at[src_v.at[j]], bufs[b], sems[b]).wait()
                        pltpu.sync_copy(bufs[b], acc.at[dst_v.at[j]], add=True)
                        nxt = j + nbuf

                        @pl.when(nxt < cnt)
                        def _():
                            pltpu.async_copy(
                                y_hbm.at[src_v.at[nxt]], bufs[b], sems[b])
                return 0

            lax.fori_loop(0, _cdiv(cnt, nbuf), ring, 0)
        plsc.subcore_barrier()
        pltpu.sync_copy(acc.at[pl.ds(sid * rpt, rpt)],
                        out_hbm.at[cid, pl.ds(sid * rpt, rpt)])

    return edge_kernel


def _dinv_from(degt_ref):
    deg = degt_ref[:, 0:1] + degt_ref[:, 1:2] + 1.0
    return lax.rsqrt(deg)


def _tc_first_body(x_ref, w_ref, degt_ref, y_ref):
    dinv = _dinv_from(degt_ref)
    xw = jnp.dot(x_ref[...], w_ref[...], preferred_element_type=jnp.float32)
    y_ref[...] = xw * dinv


def _tc_mid_body(aggp_ref, y_ref, degt_ref, b_ref, w_ref, out_ref):
    dinv = _dinv_from(degt_ref)
    s = aggp_ref[0] + aggp_ref[1] + y_ref[...]
    h = jax.nn.gelu(dinv * s + b_ref[...])
    out_ref[...] = jnp.dot(
        h, w_ref[...], preferred_element_type=jnp.float32) * dinv


def _tc_final_body(aggp_ref, y_ref, degt_ref, b_ref, out_ref):
    dinv = _dinv_from(degt_ref)
    s = aggp_ref[0] + aggp_ref[1] + y_ref[...]
    out_ref[...] = dinv * s + b_ref[...]


def _tc_first(x, w, degt, block_rows):
    n, d = x.shape
    grid = n // block_rows
    return pl.pallas_call(
        _tc_first_body,
        grid=(grid,),
        in_specs=[
            pl.BlockSpec((block_rows, d), lambda i: (i, 0)),
            pl.BlockSpec((d, d), lambda i: (0, 0)),
            pl.BlockSpec((block_rows, NC), lambda i: (i, 0)),
        ],
        out_specs=pl.BlockSpec((block_rows, d), lambda i: (i, 0)),
        out_shape=jax.ShapeDtypeStruct((n, d), jnp.float32),
    )(x, w, degt)


def _tc_mid(aggp, y, degt, b, w, block_rows):
    n, d = y.shape
    grid = n // block_rows
    return pl.pallas_call(
        _tc_mid_body,
        grid=(grid,),
        in_specs=[
            pl.BlockSpec((NC, block_rows, d), lambda i: (0, i, 0)),
            pl.BlockSpec((block_rows, d), lambda i: (i, 0)),
            pl.BlockSpec((block_rows, NC), lambda i: (i, 0)),
            pl.BlockSpec((1, d), lambda i: (0, 0)),
            pl.BlockSpec((d, d), lambda i: (0, 0)),
        ],
        out_specs=pl.BlockSpec((block_rows, d), lambda i: (i, 0)),
        out_shape=jax.ShapeDtypeStruct((n, d), jnp.float32),
    )(aggp, y, degt, b, w)


def _tc_final(aggp, y, degt, b, block_rows):
    n, d = y.shape
    grid = n // block_rows
    return pl.pallas_call(
        _tc_final_body,
        grid=(grid,),
        in_specs=[
            pl.BlockSpec((NC, block_rows, d), lambda i: (0, i, 0)),
            pl.BlockSpec((block_rows, d), lambda i: (i, 0)),
            pl.BlockSpec((block_rows, NC), lambda i: (i, 0)),
            pl.BlockSpec((1, d), lambda i: (0, 0)),
        ],
        out_specs=pl.BlockSpec((block_rows, d), lambda i: (i, 0)),
        out_shape=jax.ShapeDtypeStruct((n, d), jnp.float32),
    )(aggp, y, degt, b)


def kernel(x, edge_index, W1, b1, W2, b2, W3, b3):
    n, d = x.shape
    e = edge_index.shape[1]

    nchunks = _cdiv(e, NW * CHUNK)
    e_pad = NW * CHUNK * nchunks
    rpt = _cdiv(_cdiv(n, NS), CHUNK) * CHUNK
    n_pad = NS * rpt
    block_rows = 1000 if n % 1000 == 0 else n

    src = edge_index[0].astype(jnp.int32)
    dst = edge_index[1].astype(jnp.int32)
    src_r = jnp.concatenate(
        [src, jnp.zeros((e_pad - e,), jnp.int32)]).reshape(NW, nchunks, CHUNK)
    dst_r = jnp.concatenate(
        [dst, jnp.full((e_pad - e,), n, jnp.int32)]).reshape(NW, nchunks, CHUNK)

    deg_call = _make_deg_kernel(nchunks, n_pad, rpt)
    edge_call = _make_edge_kernel(n, d, nchunks, n_pad, rpt)

    degp = deg_call(dst_r)
    degt = degp.T[:n]

    b1r, b2r, b3r = (v.reshape(1, d) for v in (b1, b2, b3))

    y1 = _tc_first(x, W1, degt, block_rows)
    agg1 = edge_call(y1, src_r, dst_r)[:, :n]
    y2 = _tc_mid(agg1, y1, degt, b1r, W2, block_rows)
    agg2 = edge_call(y2, src_r, dst_r)[:, :n]
    y3 = _tc_mid(agg2, y2, degt, b2r, W3, block_rows)
    agg3 = edge_call(y3, src_r, dst_r)[:, :n]
    out = _tc_final(agg3, y3, degt, b3r, block_rows)
    return out


# speedup vs baseline: 19.9844x; 1.4654x over previous
import functools

import jax
import jax.numpy as jnp
from jax import lax
from jax.experimental import pallas as pl
from jax.experimental.pallas import tpu as pltpu
from jax.experimental.pallas import tpu_sc as plsc

NC = 2
NS = 16
NW = NC * NS
CHUNK = 128
LANES = 16


def _cdiv(a, b):
    return (a + b - 1) // b


def _make_deg_kernel(nch_max, nch0, nch1, n_pad, rpt):
    mesh = plsc.VectorSubcoreMesh(core_axis_name="c", subcore_axis_name="s")

    @functools.partial(
        pl.kernel,
        out_type=jax.ShapeDtypeStruct((NC, n_pad), jnp.float32),
        mesh=mesh,
        scratch_types=[
            pltpu.VMEM((nch_max, CHUNK), jnp.int32),
            pltpu.VMEM((CHUNK,), jnp.float32),
            pltpu.VMEM((rpt,), jnp.float32),
            pltpu.VMEM_SHARED((n_pad,), jnp.float32),
        ],
    )
    def deg_kernel(dst_hbm, out_hbm, dst_v, ones_v, zbuf, acc):
        cid = lax.axis_index("c")
        sid = lax.axis_index("s")
        wid = sid * NC + cid
        nch_me = jnp.where(cid == 0, nch0, nch1)

        def fill_z(j, _):
            zbuf[pl.ds(j * LANES, LANES)] = jnp.zeros((LANES,), jnp.float32)
            return 0

        lax.fori_loop(0, rpt // LANES, fill_z, 0)
        for j in range(CHUNK // LANES):
            ones_v[pl.ds(j * LANES, LANES)] = jnp.ones((LANES,), jnp.float32)
        pltpu.sync_copy(zbuf, acc.at[pl.ds(sid * rpt, rpt)])
        pltpu.sync_copy(dst_hbm.at[wid], dst_v)
        plsc.subcore_barrier()

        def chunk_step(c, _):
            @pl.when(c < nch_me)
            def _():
                pltpu.sync_copy(ones_v, acc.at[dst_v.at[c]], add=True)
            return 0

        lax.fori_loop(0, nch_max, chunk_step, 0)
        plsc.subcore_barrier()
        pltpu.sync_copy(acc.at[pl.ds(sid * rpt, rpt)],
                        out_hbm.at[cid, pl.ds(sid * rpt, rpt)])

    return deg_kernel


def _make_edge_kernel(n, d, nch_max, nch0, nch1, n_pad, rpt, nbuf=2, group=16):
    mesh = plsc.VectorSubcoreMesh(core_axis_name="c", subcore_axis_name="s")
    scratch = [
        pltpu.VMEM((group, CHUNK), jnp.int32),
        pltpu.VMEM((group, CHUNK), jnp.int32),
        pltpu.VMEM_SHARED((n_pad, d), jnp.float32),
    ]
    scratch += [pltpu.VMEM((CHUNK, d), jnp.float32) for _ in range(nbuf)]
    scratch += [pltpu.SemaphoreType.DMA for _ in range(nbuf)]

    @functools.partial(
        pl.kernel,
        out_type=jax.ShapeDtypeStruct((NC, n_pad, d), jnp.float32),
        mesh=mesh,
        scratch_types=scratch,
    )
    def edge_kernel(y_hbm, src_hbm, dst_hbm, out_hbm, src_v, dst_v, acc, *rest):
        bufs = rest[:nbuf]
        sems = rest[nbuf:]
        cid = lax.axis_index("c")
        sid = lax.axis_index("s")
        wid = sid * NC + cid
        nch_me = jnp.where(cid == 0, nch0, nch1)

        def zrow(r, _):
            for j in range(d // LANES):
                bufs[0][r, pl.ds(j * LANES, LANES)] = jnp.zeros(
                    (LANES,), jnp.float32)
            return 0

        lax.fori_loop(0, CHUNK, zrow, 0)
        for t in range(rpt // CHUNK):
            pltpu.sync_copy(
                bufs[0], acc.at[pl.ds(sid * rpt + t * CHUNK, CHUNK)])
        plsc.subcore_barrier()

        for s in range(_cdiv(nch_max, group)):
            lo = s * group
            cnt = min(group, nch_max - lo)
            cnt_me = jnp.clip(nch_me - lo, 0, cnt)
            pltpu.sync_copy(src_hbm.at[wid, pl.ds(lo, cnt)],
                            src_v.at[pl.ds(0, cnt)])
            pltpu.sync_copy(dst_hbm.at[wid, pl.ds(lo, cnt)],
                            dst_v.at[pl.ds(0, cnt)])
            for b in range(min(nbuf, cnt)):
                @pl.when(b < cnt_me)
                def _():
                    pltpu.async_copy(y_hbm.at[src_v.at[b]], bufs[b], sems[b])

            def ring(g, _):
                for b in range(nbuf):
                    j = g * nbuf + b

                    @pl.when(j < cnt_me)
                    def _():
                        pltpu.make_async_copy(
                            y_hbm.at[src_v.at[j]], bufs[b], sems[b]).wait()
                        pltpu.sync_copy(bufs[b], acc.at[dst_v.at[j]], add=True)
                        nxt = j + nbuf

                        @pl.when(nxt < cnt_me)
                        def _():
                            pltpu.async_copy(
                                y_hbm.at[src_v.at[nxt]], bufs[b], sems[b])
                return 0

            lax.fori_loop(0, _cdiv(cnt, nbuf), ring, 0)
        plsc.subcore_barrier()
        pltpu.sync_copy(acc.at[pl.ds(sid * rpt, rpt)],
                        out_hbm.at[cid, pl.ds(sid * rpt, rpt)])

    return edge_kernel


def _dinv_from(degt_ref):
    deg = degt_ref[:, 0:1] + degt_ref[:, 1:2] + 1.0
    return lax.rsqrt(deg)


def _tc_first_body(x_ref, w_ref, degt_ref, y_ref):
    dinv = _dinv_from(degt_ref)
    xw = jnp.dot(x_ref[...], w_ref[...], preferred_element_type=jnp.float32)
    y_ref[...] = xw * dinv


def _tc_mid_body(aggp_ref, y_ref, degt_ref, b_ref, w_ref, out_ref):
    dinv = _dinv_from(degt_ref)
    s = aggp_ref[0] + aggp_ref[1] + y_ref[...]
    h = jax.nn.gelu(dinv * s + b_ref[...])
    out_ref[...] = jnp.dot(
        h, w_ref[...], preferred_element_type=jnp.float32) * dinv


def _tc_final_body(aggp_ref, y_ref, degt_ref, b_ref, out_ref):
    dinv = _dinv_from(degt_ref)
    s = aggp_ref[0] + aggp_ref[1] + y_ref[...]
    out_ref[...] = dinv * s + b_ref[...]


def _tc_first(x, w, degt, block_rows):
    n, d = x.shape
    grid = n // block_rows
    return pl.pallas_call(
        _tc_first_body,
        grid=(grid,),
        in_specs=[
            pl.BlockSpec((block_rows, d), lambda i: (i, 0)),
            pl.BlockSpec((d, d), lambda i: (0, 0)),
            pl.BlockSpec((block_rows, NC), lambda i: (i, 0)),
        ],
        out_specs=pl.BlockSpec((block_rows, d), lambda i: (i, 0)),
        out_shape=jax.ShapeDtypeStruct((n, d), jnp.float32),
    )(x, w, degt)


def _tc_mid(aggp, y, degt, b, w, block_rows):
    n, d = y.shape
    grid = n // block_rows
    return pl.pallas_call(
        _tc_mid_body,
        grid=(grid,),
        in_specs=[
            pl.BlockSpec((NC, block_rows, d), lambda i: (0, i, 0)),
            pl.BlockSpec((block_rows, d), lambda i: (i, 0)),
            pl.BlockSpec((block_rows, NC), lambda i: (i, 0)),
            pl.BlockSpec((1, d), lambda i: (0, 0)),
            pl.BlockSpec((d, d), lambda i: (0, 0)),
        ],
        out_specs=pl.BlockSpec((block_rows, d), lambda i: (i, 0)),
        out_shape=jax.ShapeDtypeStruct((n, d), jnp.float32),
    )(aggp, y, degt, b, w)


def _tc_final(aggp, y, degt, b, block_rows):
    n, d = y.shape
    grid = n // block_rows
    return pl.pallas_call(
        _tc_final_body,
        grid=(grid,),
        in_specs=[
            pl.BlockSpec((NC, block_rows, d), lambda i: (0, i, 0)),
            pl.BlockSpec((block_rows, d), lambda i: (i, 0)),
            pl.BlockSpec((block_rows, NC), lambda i: (i, 0)),
            pl.BlockSpec((1, d), lambda i: (0, 0)),
        ],
        out_specs=pl.BlockSpec((block_rows, d), lambda i: (i, 0)),
        out_shape=jax.ShapeDtypeStruct((n, d), jnp.float32),
    )(aggp, y, degt, b)


def kernel(x, edge_index, W1, b1, W2, b2, W3, b3):
    n, d = x.shape
    e = edge_index.shape[1]

    rpt = _cdiv(_cdiv(n, NS), CHUNK) * CHUNK
    n_pad = NS * rpt
    block_rows = 1000 if n % 1000 == 0 else n

    pair = _cdiv(_cdiv(e, CHUNK), NS)
    nch0 = max(1, min(pair - 1, round(pair * 0.69)))
    nch1 = pair - nch0
    nch_max = max(nch0, nch1)
    e_pool = NS * pair * CHUNK

    def _layout(idx, pad_val):
        pool = jnp.concatenate(
            [idx, jnp.full((e_pool - e,), pad_val, jnp.int32)]
        ).reshape(NS, pair, CHUNK)
        part0 = pool[:, :nch0]
        part1 = jnp.concatenate(
            [pool[:, nch0:],
             jnp.full((NS, nch_max - nch1, CHUNK), pad_val, jnp.int32)],
            axis=1)
        if part0.shape[1] < nch_max:
            part0 = jnp.concatenate(
                [part0,
                 jnp.full((NS, nch_max - nch0, CHUNK), pad_val, jnp.int32)],
                axis=1)
        return jnp.stack([part0, part1], axis=1).reshape(NW, nch_max, CHUNK)

    src_r = _layout(edge_index[0].astype(jnp.int32), 0)
    dst_r = _layout(edge_index[1].astype(jnp.int32), n)

    deg_call = _make_deg_kernel(nch_max, nch0, nch1, n_pad, rpt)
    edge_call = _make_edge_kernel(n, d, nch_max, nch0, nch1, n_pad, rpt)

    degp = deg_call(dst_r)
    degt = degp.T[:n]

    b1r, b2r, b3r = (v.reshape(1, d) for v in (b1, b2, b3))

    y1 = _tc_first(x, W1, degt, block_rows)
    agg1 = edge_call(y1, src_r, dst_r)[:, :n]
    y2 = _tc_mid(agg1, y1, degt, b1r, W2, block_rows)
    agg2 = edge_call(y2, src_r, dst_r)[:, :n]
    y3 = _tc_mid(agg2, y2, degt, b2r, W3, block_rows)
    agg3 = edge_call(y3, src_r, dst_r)[:, :n]
    out = _tc_final(agg3, y3, degt, b3r, block_rows)
    return out


# speedup vs baseline: 20.5014x; 1.0259x over previous
import functools

import jax
import jax.numpy as jnp
from jax import lax
from jax.experimental import pallas as pl
from jax.experimental.pallas import tpu as pltpu
from jax.experimental.pallas import tpu_sc as plsc

NC = 2
NS = 16
NW = NC * NS
CHUNK = 128
LANES = 16


def _cdiv(a, b):
    return (a + b - 1) // b


def _make_deg_kernel(nch_max, nch0, nch1, n_pad, rpt):
    mesh = plsc.VectorSubcoreMesh(core_axis_name="c", subcore_axis_name="s")

    @functools.partial(
        pl.kernel,
        out_type=jax.ShapeDtypeStruct((NC, n_pad), jnp.float32),
        mesh=mesh,
        scratch_types=[
            pltpu.VMEM((nch_max, CHUNK), jnp.int32),
            pltpu.VMEM((CHUNK,), jnp.float32),
            pltpu.VMEM((rpt,), jnp.float32),
            pltpu.VMEM_SHARED((n_pad,), jnp.float32),
        ],
    )
    def deg_kernel(dst_hbm, out_hbm, dst_v, ones_v, zbuf, acc):
        cid = lax.axis_index("c")
        sid = lax.axis_index("s")
        wid = sid * NC + cid
        nch_me = jnp.where(cid == 0, nch0, nch1)

        def fill_z(j, _):
            zbuf[pl.ds(j * LANES, LANES)] = jnp.zeros((LANES,), jnp.float32)
            return 0

        lax.fori_loop(0, rpt // LANES, fill_z, 0)
        for j in range(CHUNK // LANES):
            ones_v[pl.ds(j * LANES, LANES)] = jnp.ones((LANES,), jnp.float32)
        pltpu.sync_copy(zbuf, acc.at[pl.ds(sid * rpt, rpt)])
        pltpu.sync_copy(dst_hbm.at[wid], dst_v)
        plsc.subcore_barrier()

        def chunk_step(c, _):
            @pl.when(c < nch_me)
            def _():
                pltpu.sync_copy(ones_v, acc.at[dst_v.at[c]], add=True)
            return 0

        lax.fori_loop(0, nch_max, chunk_step, 0)
        plsc.subcore_barrier()
        pltpu.sync_copy(acc.at[pl.ds(sid * rpt, rpt)],
                        out_hbm.at[cid, pl.ds(sid * rpt, rpt)])

    return deg_kernel


def _make_edge_kernel(n, d, nch_max, nch0, nch1, n_pad, rpt, nbuf=2, group=16):
    mesh = plsc.VectorSubcoreMesh(core_axis_name="c", subcore_axis_name="s")
    scratch = [
        pltpu.VMEM((group, CHUNK), jnp.int32),
        pltpu.VMEM((group, CHUNK), jnp.int32),
        pltpu.VMEM_SHARED((n_pad, d), jnp.float32),
    ]
    scratch += [pltpu.VMEM((CHUNK, d), jnp.float32) for _ in range(nbuf)]
    scratch += [pltpu.SemaphoreType.DMA for _ in range(nbuf)]

    @functools.partial(
        pl.kernel,
        out_type=jax.ShapeDtypeStruct((NC, n_pad, d), jnp.float32),
        mesh=mesh,
        scratch_types=scratch,
    )
    def edge_kernel(y_hbm, src_hbm, dst_hbm, out_hbm, src_v, dst_v, acc, *rest):
        bufs = rest[:nbuf]
        sems = rest[nbuf:]
        cid = lax.axis_index("c")
        sid = lax.axis_index("s")
        wid = sid * NC + cid
        nch_me = jnp.where(cid == 0, nch0, nch1)

        def zrow(r, _):
            for j in range(d // LANES):
                bufs[0][r, pl.ds(j * LANES, LANES)] = jnp.zeros(
                    (LANES,), jnp.float32)
            return 0

        lax.fori_loop(0, CHUNK, zrow, 0)
        for t in range(rpt // CHUNK):
            pltpu.sync_copy(
                bufs[0], acc.at[pl.ds(sid * rpt + t * CHUNK, CHUNK)])
        plsc.subcore_barrier()

        for s in range(_cdiv(nch_max, group)):
            lo = s * group
            cnt = min(group, nch_max - lo)
            cnt_me = jnp.clip(nch_me - lo, 0, cnt)
            pltpu.sync_copy(src_hbm.at[wid, pl.ds(lo, cnt)],
                            src_v.at[pl.ds(0, cnt)])
            pltpu.sync_copy(dst_hbm.at[wid, pl.ds(lo, cnt)],
                            dst_v.at[pl.ds(0, cnt)])
            for b in range(min(nbuf, cnt)):
                @pl.when(b < cnt_me)
                def _():
                    pltpu.async_copy(y_hbm.at[src_v.at[b]], bufs[b], sems[b])

            def ring(g, _):
                for b in range(nbuf):
                    j = g * nbuf + b

                    @pl.when(j < cnt_me)
                    def _():
                        pltpu.make_async_copy(
                            y_hbm.at[src_v.at[j]], bufs[b], sems[b]).wait()
                        pltpu.sync_copy(bufs[b], acc.at[dst_v.at[j]], add=True)
                        nxt = j + nbuf

                        @pl.when(nxt < cnt_me)
                        def _():
                            pltpu.async_copy(
                                y_hbm.at[src_v.at[nxt]], bufs[b], sems[b])
                return 0

            lax.fori_loop(0, _cdiv(cnt, nbuf), ring, 0)
        plsc.subcore_barrier()
        pltpu.sync_copy(acc.at[pl.ds(sid * rpt, rpt)],
                        out_hbm.at[cid, pl.ds(sid * rpt, rpt)])

    return edge_kernel


def _dinv_from(degt_ref):
    deg = degt_ref[:, 0:1] + degt_ref[:, 1:2] + 1.0
    return lax.rsqrt(deg)


def _tc_first_body(x_ref, w_ref, degt_ref, y_ref):
    dinv = _dinv_from(degt_ref)
    xw = jnp.dot(x_ref[...], w_ref[...], preferred_element_type=jnp.float32)
    y_ref[...] = xw * dinv


def _tc_mid_body(aggp_ref, y_ref, degt_ref, b_ref, w_ref, out_ref):
    dinv = _dinv_from(degt_ref)
    s = aggp_ref[0] + aggp_ref[1] + y_ref[...]
    h = jax.nn.gelu(dinv * s + b_ref[...])
    out_ref[...] = jnp.dot(
        h, w_ref[...], preferred_element_type=jnp.float32) * dinv


def _tc_final_body(aggp_ref, y_ref, degt_ref, b_ref, out_ref):
    dinv = _dinv_from(degt_ref)
    s = aggp_ref[0] + aggp_ref[1] + y_ref[...]
    out_ref[...] = dinv * s + b_ref[...]


def _tc_first(x, w, degt, block_rows):
    n, d = x.shape
    grid = n // block_rows
    return pl.pallas_call(
        _tc_first_body,
        grid=(grid,),
        in_specs=[
            pl.BlockSpec((block_rows, d), lambda i: (i, 0)),
            pl.BlockSpec((d, d), lambda i: (0, 0)),
            pl.BlockSpec((block_rows, NC), lambda i: (i, 0)),
        ],
        out_specs=pl.BlockSpec((block_rows, d), lambda i: (i, 0)),
        out_shape=jax.ShapeDtypeStruct((n, d), jnp.float32),
    )(x, w, degt)


def _tc_mid(aggp, y, degt, b, w, block_rows):
    n, d = y.shape
    grid = n // block_rows
    return pl.pallas_call(
        _tc_mid_body,
        grid=(grid,),
        in_specs=[
            pl.BlockSpec((NC, block_rows, d), lambda i: (0, i, 0)),
            pl.BlockSpec((block_rows, d), lambda i: (i, 0)),
            pl.BlockSpec((block_rows, NC), lambda i: (i, 0)),
            pl.BlockSpec((1, d), lambda i: (0, 0)),
            pl.BlockSpec((d, d), lambda i: (0, 0)),
        ],
        out_specs=pl.BlockSpec((block_rows, d), lambda i: (i, 0)),
        out_shape=jax.ShapeDtypeStruct((n, d), jnp.float32),
    )(aggp, y, degt, b, w)


def _tc_final(aggp, y, degt, b, block_rows):
    n, d = y.shape
    grid = n // block_rows
    return pl.pallas_call(
        _tc_final_body,
        grid=(grid,),
        in_specs=[
            pl.BlockSpec((NC, block_rows, d), lambda i: (0, i, 0)),
            pl.BlockSpec((block_rows, d), lambda i: (i, 0)),
            pl.BlockSpec((block_rows, NC), lambda i: (i, 0)),
            pl.BlockSpec((1, d), lambda i: (0, 0)),
        ],
        out_specs=pl.BlockSpec((block_rows, d), lambda i: (i, 0)),
        out_shape=jax.ShapeDtypeStruct((n, d), jnp.float32),
    )(aggp, y, degt, b)


def kernel(x, edge_index, W1, b1, W2, b2, W3, b3):
    n, d = x.shape
    e = edge_index.shape[1]

    rpt = _cdiv(_cdiv(n, NS), CHUNK) * CHUNK
    n_pad = NS * rpt
    block_rows = 1000 if n % 1000 == 0 else n

    pair = _cdiv(_cdiv(e, CHUNK), NS)
    nch0 = max(1, min(pair - 1, round(pair * 0.69)))
    nch1 = pair - nch0
    nch_max = max(nch0, nch1)
    e_pool = NS * pair * CHUNK

    def _layout(idx, pad_val):
        pool = jnp.concatenate(
            [idx, jnp.full((e_pool - e,), pad_val, jnp.int32)]
        ).reshape(NS, pair, CHUNK)
        part0 = pool[:, :nch0]
        part1 = jnp.concatenate(
            [pool[:, nch0:],
             jnp.full((NS, nch_max - nch1, CHUNK), pad_val, jnp.int32)],
            axis=1)
        if part0.shape[1] < nch_max:
            part0 = jnp.concatenate(
                [part0,
                 jnp.full((NS, nch_max - nch0, CHUNK), pad_val, jnp.int32)],
                axis=1)
        return jnp.stack([part0, part1], axis=1).reshape(NW, nch_max, CHUNK)

    src_r = _layout(edge_index[0].astype(jnp.int32), 0)
    dst_r = _layout(edge_index[1].astype(jnp.int32), n)

    deg_call = _make_deg_kernel(nch_max, nch0, nch1, n_pad, rpt)
    edge_call = _make_edge_kernel(n, d, nch_max, nch0, nch1, n_pad, rpt)

    degp = deg_call(dst_r)
    degt = degp.T

    b1r, b2r, b3r = (v.reshape(1, d) for v in (b1, b2, b3))

    y1 = _tc_first(x, W1, degt, block_rows)
    agg1 = edge_call(y1, src_r, dst_r)
    y2 = _tc_mid(agg1, y1, degt, b1r, W2, block_rows)
    agg2 = edge_call(y2, src_r, dst_r)
    y3 = _tc_mid(agg2, y2, degt, b2r, W3, block_rows)
    agg3 = edge_call(y3, src_r, dst_r)
    out = _tc_final(agg3, y3, degt, b3r, block_rows)
    return out


# speedup vs baseline: 22.2910x; 1.0873x over previous
import functools

import jax
import jax.numpy as jnp
from jax import lax
from jax.experimental import pallas as pl
from jax.experimental.pallas import tpu as pltpu
from jax.experimental.pallas import tpu_sc as plsc

NC = 2
NS = 16
NW = NC * NS
CHUNK = 112
NBUF = 3
LANES = 16


def _cdiv(a, b):
    return (a + b - 1) // b


def _make_deg_kernel(nch_max, nch0, nch1, n_pad, rpt):
    mesh = plsc.VectorSubcoreMesh(core_axis_name="c", subcore_axis_name="s")

    @functools.partial(
        pl.kernel,
        out_type=jax.ShapeDtypeStruct((NC, n_pad), jnp.float32),
        mesh=mesh,
        scratch_types=[
            pltpu.VMEM((nch_max, CHUNK), jnp.int32),
            pltpu.VMEM((CHUNK,), jnp.float32),
            pltpu.VMEM((rpt,), jnp.float32),
            pltpu.VMEM_SHARED((n_pad,), jnp.float32),
        ],
    )
    def deg_kernel(dst_hbm, out_hbm, dst_v, ones_v, zbuf, acc):
        cid = lax.axis_index("c")
        sid = lax.axis_index("s")
        wid = sid * NC + cid
        nch_me = jnp.where(cid == 0, nch0, nch1)

        def fill_z(j, _):
            zbuf[pl.ds(j * LANES, LANES)] = jnp.zeros((LANES,), jnp.float32)
            return 0

        lax.fori_loop(0, rpt // LANES, fill_z, 0)
        for j in range(CHUNK // LANES):
            ones_v[pl.ds(j * LANES, LANES)] = jnp.ones((LANES,), jnp.float32)
        pltpu.sync_copy(zbuf, acc.at[pl.ds(sid * rpt, rpt)])
        pltpu.sync_copy(dst_hbm.at[wid], dst_v)
        plsc.subcore_barrier()

        def chunk_step(c, _):
            @pl.when(c < nch_me)
            def _():
                pltpu.sync_copy(ones_v, acc.at[dst_v.at[c]], add=True)
            return 0

        lax.fori_loop(0, nch_max, chunk_step, 0)
        plsc.subcore_barrier()
        pltpu.sync_copy(acc.at[pl.ds(sid * rpt, rpt)],
                        out_hbm.at[cid, pl.ds(sid * rpt, rpt)])

    return deg_kernel


def _make_edge_kernel(n, d, nch_max, nch0, nch1, n_pad, rpt, nbuf=NBUF,
                      group=16):
    mesh = plsc.VectorSubcoreMesh(core_axis_name="c", subcore_axis_name="s")
    scratch = [
        pltpu.VMEM((group, CHUNK), jnp.int32),
        pltpu.VMEM((group, CHUNK), jnp.int32),
        pltpu.VMEM_SHARED((n_pad, d), jnp.float32),
    ]
    scratch += [pltpu.VMEM((CHUNK, d), jnp.float32) for _ in range(nbuf)]
    scratch += [pltpu.SemaphoreType.DMA for _ in range(nbuf)]

    @functools.partial(
        pl.kernel,
        out_type=jax.ShapeDtypeStruct((NC, n_pad, d), jnp.float32),
        mesh=mesh,
        scratch_types=scratch,
    )
    def edge_kernel(y_hbm, src_hbm, dst_hbm, out_hbm, src_v, dst_v, acc, *rest):
        bufs = rest[:nbuf]
        sems = rest[nbuf:]
        cid = lax.axis_index("c")
        sid = lax.axis_index("s")
        wid = sid * NC + cid
        nch_me = jnp.where(cid == 0, nch0, nch1)

        def zrow(r, _):
            for j in range(d // LANES):
                bufs[0][r, pl.ds(j * LANES, LANES)] = jnp.zeros(
                    (LANES,), jnp.float32)
            return 0

        lax.fori_loop(0, CHUNK, zrow, 0)
        for t in range(rpt // CHUNK):
            pltpu.sync_copy(
                bufs[0], acc.at[pl.ds(sid * rpt + t * CHUNK, CHUNK)])
        rem = rpt % CHUNK
        if rem:
            pltpu.sync_copy(
                bufs[0].at[pl.ds(0, rem)],
                acc.at[pl.ds(sid * rpt + (rpt // CHUNK) * CHUNK, rem)])
        plsc.subcore_barrier()

        for s in range(_cdiv(nch_max, group)):
            lo = s * group
            cnt = min(group, nch_max - lo)
            cnt_me = jnp.clip(nch_me - lo, 0, cnt)
            pltpu.sync_copy(src_hbm.at[wid, pl.ds(lo, cnt)],
                            src_v.at[pl.ds(0, cnt)])
            pltpu.sync_copy(dst_hbm.at[wid, pl.ds(lo, cnt)],
                            dst_v.at[pl.ds(0, cnt)])
            for b in range(min(nbuf, cnt)):
                @pl.when(b < cnt_me)
                def _():
                    pltpu.async_copy(y_hbm.at[src_v.at[b]], bufs[b], sems[b])

            def ring(g, _):
                for b in range(nbuf):
                    j = g * nbuf + b

                    @pl.when(j < cnt_me)
                    def _():
                        pltpu.make_async_copy(
                            y_hbm.at[src_v.at[j]], bufs[b], sems[b]).wait()
                        pltpu.sync_copy(bufs[b], acc.at[dst_v.at[j]], add=True)
                        nxt = j + nbuf

                        @pl.when(nxt < cnt_me)
                        def _():
                            pltpu.async_copy(
                                y_hbm.at[src_v.at[nxt]], bufs[b], sems[b])
                return 0

            lax.fori_loop(0, _cdiv(cnt, nbuf), ring, 0)
        plsc.subcore_barrier()
        pltpu.sync_copy(acc.at[pl.ds(sid * rpt, rpt)],
                        out_hbm.at[cid, pl.ds(sid * rpt, rpt)])

    return edge_kernel


def _dinv_from(degt_ref):
    deg = degt_ref[:, 0:1] + degt_ref[:, 1:2] + 1.0
    return lax.rsqrt(deg)


def _tc_first_body(x_ref, w_ref, degt_ref, y_ref):
    dinv = _dinv_from(degt_ref)
    xw = jnp.dot(x_ref[...], w_ref[...], preferred_element_type=jnp.float32)
    y_ref[...] = xw * dinv


def _tc_mid_body(aggp_ref, y_ref, degt_ref, b_ref, w_ref, out_ref):
    dinv = _dinv_from(degt_ref)
    s = aggp_ref[0] + aggp_ref[1] + y_ref[...]
    h = jax.nn.gelu(dinv * s + b_ref[...])
    out_ref[...] = jnp.dot(
        h, w_ref[...], preferred_element_type=jnp.float32) * dinv


def _tc_final_body(aggp_ref, y_ref, degt_ref, b_ref, out_ref):
    dinv = _dinv_from(degt_ref)
    s = aggp_ref[0] + aggp_ref[1] + y_ref[...]
    out_ref[...] = dinv * s + b_ref[...]


def _tc_first(x, w, degt, block_rows):
    n, d = x.shape
    grid = n // block_rows
    return pl.pallas_call(
        _tc_first_body,
        grid=(grid,),
        in_specs=[
            pl.BlockSpec((block_rows, d), lambda i: (i, 0)),
            pl.BlockSpec((d, d), lambda i: (0, 0)),
            pl.BlockSpec((block_rows, NC), lambda i: (i, 0)),
        ],
        out_specs=pl.BlockSpec((block_rows, d), lambda i: (i, 0)),
        out_shape=jax.ShapeDtypeStruct((n, d), jnp.float32),
    )(x, w, degt)


def _tc_mid(aggp, y, degt, b, w, block_rows):
    n, d = y.shape
    grid = n // block_rows
    return pl.pallas_call(
        _tc_mid_body,
        grid=(grid,),
        in_specs=[
            pl.BlockSpec((NC, block_rows, d), lambda i: (0, i, 0)),
            pl.BlockSpec((block_rows, d), lambda i: (i, 0)),
            pl.BlockSpec((block_rows, NC), lambda i: (i, 0)),
            pl.BlockSpec((1, d), lambda i: (0, 0)),
            pl.BlockSpec((d, d), lambda i: (0, 0)),
        ],
        out_specs=pl.BlockSpec((block_rows, d), lambda i: (i, 0)),
        out_shape=jax.ShapeDtypeStruct((n, d), jnp.float32),
    )(aggp, y, degt, b, w)


def _tc_final(aggp, y, degt, b, block_rows):
    n, d = y.shape
    grid = n // block_rows
    return pl.pallas_call(
        _tc_final_body,
        grid=(grid,),
        in_specs=[
            pl.BlockSpec((NC, block_rows, d), lambda i: (0, i, 0)),
            pl.BlockSpec((block_rows, d), lambda i: (i, 0)),
            pl.BlockSpec((block_rows, NC), lambda i: (i, 0)),
            pl.BlockSpec((1, d), lambda i: (0, 0)),
        ],
        out_specs=pl.BlockSpec((block_rows, d), lambda i: (i, 0)),
        out_shape=jax.ShapeDtypeStruct((n, d), jnp.float32),
    )(aggp, y, degt, b)


def kernel(x, edge_index, W1, b1, W2, b2, W3, b3):
    n, d = x.shape
    e = edge_index.shape[1]

    rpt = _cdiv(_cdiv(n, NS), 8) * 8
    n_pad = NS * rpt
    rpt_deg = _cdiv(_cdiv(n, NS), LANES) * LANES
    n_pad_deg = NS * rpt_deg
    block_rows = 1000 if n % 1000 == 0 else n

    pair = _cdiv(_cdiv(e, CHUNK), NS)
    nch0 = max(1, min(pair - 1, round(pair * 0.69)))
    nch1 = pair - nch0
    nch_max = _cdiv(max(nch0, nch1), 8) * 8
    e_pool = NS * pair * CHUNK

    def _layout(idx, pad_val):
        pool = jnp.concatenate(
            [idx, jnp.full((e_pool - e,), pad_val, jnp.int32)]
        ).reshape(NS, pair, CHUNK)
        part0 = pool[:, :nch0]
        part1 = jnp.concatenate(
            [pool[:, nch0:],
             jnp.full((NS, nch_max - nch1, CHUNK), pad_val, jnp.int32)],
            axis=1)
        if part0.shape[1] < nch_max:
            part0 = jnp.concatenate(
                [part0,
                 jnp.full((NS, nch_max - nch0, CHUNK), pad_val, jnp.int32)],
                axis=1)
        return jnp.stack([part0, part1], axis=1).reshape(NW, nch_max, CHUNK)

    src_r = _layout(edge_index[0].astype(jnp.int32), 0)
    dst_r = _layout(edge_index[1].astype(jnp.int32), n)

    deg_call = _make_deg_kernel(nch_max, nch0, nch1, n_pad_deg, rpt_deg)
    edge_call = _make_edge_kernel(n, d, nch_max, nch0, nch1, n_pad, rpt)

    degp = deg_call(dst_r)
    degt = degp.T

    b1r, b2r, b3r = (v.reshape(1, d) for v in (b1, b2, b3))

    y1 = _tc_first(x, W1, degt, block_rows)
    agg1 = edge_call(y1, src_r, dst_r)
    y2 = _tc_mid(agg1, y1, degt, b1r, W2, block_rows)
    agg2 = edge_call(y2, src_r, dst_r)
    y3 = _tc_mid(agg2, y2, degt, b2r, W3, block_rows)
    agg3 = edge_call(y3, src_r, dst_r)
    out = _tc_final(agg3, y3, degt, b3r, block_rows)
    return out
